# baseline (speedup 1.0000x reference)
import functools

import jax
import jax.numpy as jnp
from jax import lax
from jax.experimental import pallas as pl
from jax.experimental.pallas import tpu as pltpu
from jax.experimental.pallas import tpu_sc as plsc

N = 10000
E = 320000
D = 128
EMB = 128
L = 5
DE = 16
C = 10
G = 128

NW = 32
ROW = 128
RPW = 80
EPAD = NW * RPW * ROW
NITER = RPW // 4
NPAD = 10008
CHUNK = 624
LCHUNK = NPAD - 15 * CHUNK

NB = 1000
EB = 4096


def _sc_agg_body(woff, h_hbm, e_hbm, idx_hbm, out_hbm,
                 iA, iB, ebuf, hb0, hb1, agg_sh,
                 sem_e, sem_h0, sem_h1, sem_s0, sem_s1, sem_iA, sem_iB):
    s = lax.axis_index("s")
    w = s + woff
    pbase = w * (2 * RPW)
    erow0 = w * (RPW * ROW)

    def _zb(i, carry):
        for j in range(8):
            ebuf[i, pl.ds(j * 16, 16)] = jnp.zeros((16,), jnp.float32)
        return carry
    lax.fori_loop(0, ROW, _zb, 0)
    base = s * CHUNK

    def _zero_rows(nrows):
        for off in range(0, nrows - ROW + 1, ROW):
            pltpu.sync_copy(ebuf, agg_sh.at[pl.ds(base + off, ROW)])
        rem = nrows % ROW
        if rem:
            pltpu.sync_copy(ebuf.at[pl.ds(0, rem)],
                            agg_sh.at[pl.ds(base + nrows - rem, rem)])

    @pl.when(s < 15)
    def _():
        _zero_rows(CHUNK)

    @pl.when(s == 15)
    def _():
        _zero_rows(LCHUNK)
    plsc.subcore_barrier()

    def _relu_add(dst_b, src_b):
        def _cb(i2, c2):
            i = i2 * 4
            for k in range(4):
                for j in range(8):
                    sl = pl.ds(j * 16, 16)
                    dst_b[i + k, sl] = jnp.maximum(
                        dst_b[i + k, sl] + src_b[i + k, sl], 0.0)
            return c2
        lax.fori_loop(0, ROW // 4, _cb, 0)

    pltpu.sync_copy(idx_hbm.at[pl.ds(pbase, 4)], iA)
    pltpu.async_copy(e_hbm.at[pl.ds(erow0, ROW)], ebuf, sem_e)
    pltpu.async_copy(h_hbm.at[iA.at[0]], hb0, sem_h0)

    def _quad(m, carry):
        a = 4 * m
        erow_a = erow0 + a * ROW
        pbA = pbase + 8 * m

        @pl.when(m > 0)
        def _():
            pltpu.make_async_copy(hb1, agg_sh.at[iB.at[3]], sem_s1).wait()
        pltpu.async_copy(idx_hbm.at[pl.ds(pbA + 4, 4)], iB, sem_iB)
        pltpu.async_copy(h_hbm.at[iA.at[1]], hb1, sem_h1)

        pltpu.make_async_copy(e_hbm.at[pl.ds(erow_a, ROW)], ebuf, sem_e).wait()
        pltpu.make_async_copy(h_hbm.at[iA.at[0]], hb0, sem_h0).wait()
        _relu_add(hb0, ebuf)
        pltpu.async_copy(e_hbm.at[pl.ds(erow_a + ROW, ROW)], ebuf, sem_e)
        pltpu.async_copy(hb0, agg_sh.at[iA.at[2]], sem_s0, add=True)

        pltpu.make_async_copy(e_hbm.at[pl.ds(erow_a + ROW, ROW)], ebuf,
                              sem_e).wait()
        pltpu.make_async_copy(h_hbm.at[iA.at[1]], hb1, sem_h1).wait()
        _relu_add(hb1, ebuf)
        pltpu.make_async_copy(hb0, agg_sh.at[iA.at[2]], sem_s0).wait()

        pltpu.make_async_copy(idx_hbm.at[pl.ds(pbA + 4, 4)], iB, sem_iB).wait()
        pltpu.async_copy(h_hbm.at[iB.at[0]], hb0, sem_h0)
        pltpu.async_copy(e_hbm.at[pl.ds(erow_a + 2 * ROW, ROW)], ebuf, sem_e)
        pltpu.async_copy(hb1, agg_sh.at[iA.at[3]], sem_s1, add=True)

        pltpu.make_async_copy(hb1, agg_sh.at[iA.at[3]], sem_s1).wait()

        @pl.when(m < NITER - 1)
        def _():
            pltpu.async_copy(idx_hbm.at[pl.ds(pbA + 8, 4)], iA, sem_iA)
        pltpu.async_copy(h_hbm.at[iB.at[1]], hb1, sem_h1)

        pltpu.make_async_copy(e_hbm.at[pl.ds(erow_a + 2 * ROW, ROW)], ebuf,
                              sem_e).wait()
        pltpu.make_async_copy(h_hbm.at[iB.at[0]], hb0, sem_h0).wait()
        _relu_add(hb0, ebuf)
        pltpu.async_copy(e_hbm.at[pl.ds(erow_a + 3 * ROW, ROW)], ebuf, sem_e)
        pltpu.async_copy(hb0, agg_sh.at[iB.at[2]], sem_s0, add=True)

        pltpu.make_async_copy(e_hbm.at[pl.ds(erow_a + 3 * ROW, ROW)], ebuf,
                              sem_e).wait()
        pltpu.make_async_copy(h_hbm.at[iB.at[1]], hb1, sem_h1).wait()
        _relu_add(hb1, ebuf)
        pltpu.make_async_copy(hb0, agg_sh.at[iB.at[2]], sem_s0).wait()

        @pl.when(m < NITER - 1)
        def _():
            pltpu.make_async_copy(idx_hbm.at[pl.ds(pbA + 8, 4)], iA,
                                  sem_iA).wait()
            pltpu.async_copy(h_hbm.at[iA.at[0]], hb0, sem_h0)
            pltpu.async_copy(e_hbm.at[pl.ds(erow_a + 4 * ROW, ROW)], ebuf,
                             sem_e)
        pltpu.async_copy(hb1, agg_sh.at[iB.at[3]], sem_s1, add=True)
        return carry
    lax.fori_loop(0, NITER, _quad, 0)
    pltpu.make_async_copy(hb1, agg_sh.at[iB.at[3]], sem_s1).wait()

    plsc.subcore_barrier()

    @pl.when(s < 15)
    def _():
        pltpu.sync_copy(agg_sh.at[pl.ds(base, CHUNK)],
                        out_hbm.at[pl.ds(base, CHUNK)])

    @pl.when(s == 15)
    def _():
        pltpu.sync_copy(agg_sh.at[pl.ds(base, LCHUNK)],
                        out_hbm.at[pl.ds(base, LCHUNK)])


def _make_sc_agg(woff):
    return pl.kernel(
        functools.partial(_sc_agg_body, woff),
        out_type=jax.ShapeDtypeStruct((NPAD, EMB), jnp.float32),
        mesh=plsc.VectorSubcoreMesh(core_axis_name="c", subcore_axis_name="s",
                                    num_cores=1),
        scratch_types=[
        pltpu.VMEM((4, ROW), jnp.int32),
        pltpu.VMEM((4, ROW), jnp.int32),
        pltpu.VMEM((ROW, EMB), jnp.float32),
        pltpu.VMEM((ROW, EMB), jnp.float32),
        pltpu.VMEM((ROW, EMB), jnp.float32),
        pltpu.VMEM_SHARED((NPAD, EMB), jnp.float32),
        pltpu.SemaphoreType.DMA,
        pltpu.SemaphoreType.DMA,
        pltpu.SemaphoreType.DMA,
        pltpu.SemaphoreType.DMA,
        pltpu.SemaphoreType.DMA,
        pltpu.SemaphoreType.DMA,
        pltpu.SemaphoreType.DMA,
    ],
    )


_sc_agg0 = _make_sc_agg(0)
_sc_agg1 = _make_sc_agg(16)


def _enc_body(x_ref, w_ref, b_ref, g_ref, be_ref, o_ref):
    acc = jnp.dot(x_ref[...], w_ref[...], preferred_element_type=jnp.float32)
    o_ref[...] = g_ref[...] * (acc + b_ref[...]) + be_ref[...]


def _encode(x, W_enc, b_enc, g0, be0):
    return pl.pallas_call(
        _enc_body,
        grid=(N // NB,),
        in_specs=[
            pl.BlockSpec((NB, D), lambda i: (i, 0)),
            pl.BlockSpec((D, EMB), lambda i: (0, 0)),
            pl.BlockSpec((1, EMB), lambda i: (0, 0)),
            pl.BlockSpec((1, EMB), lambda i: (0, 0)),
            pl.BlockSpec((1, EMB), lambda i: (0, 0)),
        ],
        out_specs=pl.BlockSpec((NB, EMB), lambda i: (i, 0)),
        out_shape=jax.ShapeDtypeStruct((N, EMB), jnp.float32),
    )(x, W_enc, b_enc.reshape(1, EMB), g0.reshape(1, EMB), be0.reshape(1, EMB))


def _eemb_body(ea_ref, w_ref, b_ref, o_ref):
    acc = jnp.dot(ea_ref[...], w_ref[...], preferred_element_type=jnp.float32)
    o_ref[...] = acc + b_ref[...]


def _edge_embed(ea_pad, W_l, b_l):
    return pl.pallas_call(
        _eemb_body,
        grid=(EPAD // EB,),
        in_specs=[
            pl.BlockSpec((EB, DE), lambda i: (i, 0)),
            pl.BlockSpec((DE, EMB), lambda i: (0, 0)),
            pl.BlockSpec((1, EMB), lambda i: (0, 0)),
        ],
        out_specs=pl.BlockSpec((EB, EMB), lambda i: (i, 0)),
        out_shape=jax.ShapeDtypeStruct((EPAD, EMB), jnp.float32),
    )(ea_pad, W_l, b_l.reshape(1, EMB))


def _node_body(do_relu, h_ref, p0_ref, p1_ref, eps_ref, w1_ref, b1_ref,
               w2_ref, b2_ref, gl_ref, bl_ref, o_ref):
    t = (1.0 + eps_ref[0, 0]) * h_ref[...] + p0_ref[...] + p1_ref[...]
    u = jnp.maximum(
        jnp.dot(t, w1_ref[...], preferred_element_type=jnp.float32)
        + b1_ref[...], 0.0)
    v = jnp.dot(u, w2_ref[...], preferred_element_type=jnp.float32) + b2_ref[...]
    t = gl_ref[...] * v + bl_ref[...]
    if do_relu:
        t = jnp.maximum(t, 0.0)
    o_ref[...] = t


def _node_mlp(h, p0, p1, eps_l, W1_l, b1_l, W2_l, b2_l, gl_l, bl_l, do_relu):
    return pl.pallas_call(
        functools.partial(_node_body, do_relu),
        grid=(N // NB,),
        in_specs=[
            pl.BlockSpec((NB, EMB), lambda i: (i, 0)),
            pl.BlockSpec((NB, EMB), lambda i: (i, 0)),
            pl.BlockSpec((NB, EMB), lambda i: (i, 0)),
            pl.BlockSpec((1, 1), lambda i: (0, 0)),
            pl.BlockSpec((EMB, 2 * EMB), lambda i: (0, 0)),
            pl.BlockSpec((1, 2 * EMB), lambda i: (0, 0)),
            pl.BlockSpec((2 * EMB, EMB), lambda i: (0, 0)),
            pl.BlockSpec((1, EMB), lambda i: (0, 0)),
            pl.BlockSpec((1, EMB), lambda i: (0, 0)),
            pl.BlockSpec((1, EMB), lambda i: (0, 0)),
        ],
        out_specs=pl.BlockSpec((NB, EMB), lambda i: (i, 0)),
        out_shape=jax.ShapeDtypeStruct((N, EMB), jnp.float32),
    )(h, p0, p1, eps_l.reshape(1, 1), W1_l, b1_l.reshape(1, 2 * EMB),
      W2_l, b2_l.reshape(1, EMB), gl_l.reshape(1, EMB), bl_l.reshape(1, EMB))


def _readout_body(h_ref, b_ref, wp1_ref, bp1_ref, wp2_ref, bp2_ref, o_ref,
                  hg_ref):
    i = pl.program_id(0)

    @pl.when(i == 0)
    def _():
        hg_ref[...] = jnp.zeros_like(hg_ref)

    bblk = b_ref[0, 0, :]
    onehot = (lax.broadcasted_iota(jnp.int32, (G, NB), 0)
              == bblk[None, :]).astype(jnp.float32)
    hg_ref[...] += jnp.dot(onehot, h_ref[...],
                           preferred_element_type=jnp.float32)

    @pl.when(i == (N // NB) - 1)
    def _():
        hg = hg_ref[...]
        z = jax.nn.sigmoid(
            jnp.dot(hg, wp1_ref[...], preferred_element_type=jnp.float32)
            + bp1_ref[...])
        o_ref[...] = jnp.dot(z, wp2_ref[...],
                             preferred_element_type=jnp.float32) + bp2_ref[...]


def _readout(h, batch3d, Wp1, bp1, Wp2, bp2):
    return pl.pallas_call(
        _readout_body,
        grid=(N // NB,),
        in_specs=[
            pl.BlockSpec((NB, EMB), lambda i: (i, 0)),
            pl.BlockSpec((1, 1, NB), lambda i: (i, 0, 0)),
            pl.BlockSpec((EMB, EMB), lambda i: (0, 0)),
            pl.BlockSpec((1, EMB), lambda i: (0, 0)),
            pl.BlockSpec((EMB, C), lambda i: (0, 0)),
            pl.BlockSpec((1, C), lambda i: (0, 0)),
        ],
        out_specs=pl.BlockSpec((G, C), lambda i: (0, 0)),
        out_shape=jax.ShapeDtypeStruct((G, C), jnp.float32),
        scratch_shapes=[pltpu.VMEM((G, EMB), jnp.float32)],
    )(h, batch3d, Wp1, bp1.reshape(1, EMB), Wp2, bp2.reshape(1, C))


def kernel(x, edge_index, edge_attr, batch, W_enc, b_enc, g0, be0,
           W_edge, b_edge, eps, W1, b1, W2, b2, gl, bl, Wp1, bp1, Wp2, bp2):
    pad = EPAD - E
    src2d = jnp.concatenate(
        [edge_index[0], jnp.zeros((pad,), jnp.int32)]).reshape(EPAD // ROW, ROW)
    dst2d = jnp.concatenate(
        [edge_index[1], jnp.full((pad,), N, jnp.int32)]).reshape(EPAD // ROW, ROW)
    npair_all = EPAD // ROW // 2
    idx_packed = jnp.concatenate(
        [src2d.reshape(npair_all, 2, ROW), dst2d.reshape(npair_all, 2, ROW)],
        axis=1).reshape(4 * npair_all, ROW)
    ea_pad = jnp.concatenate(
        [edge_attr, jnp.zeros((pad, DE), jnp.float32)])
    batch3d = batch.reshape(N // NB, 1, NB)

    h = _encode(x, W_enc, b_enc, g0, be0)
    e_embs = [_edge_embed(ea_pad, W_edge[l], b_edge[l]) for l in range(L)]
    for l in range(L):
        p0 = _sc_agg0(h, e_embs[l], idx_packed)
        p1 = _sc_agg1(h, e_embs[l], idx_packed)
        h = _node_mlp(h, p0, p1, eps[l], W1[l], b1[l], W2[l], b2[l],
                      gl[l], bl[l], do_relu=(l < L - 1))
    return _readout(h, batch3d, Wp1, bp1, Wp2, bp2)


# speedup vs baseline: 1.2774x; 1.2774x over previous
import functools

import jax
import jax.numpy as jnp
from jax import lax
from jax.experimental import pallas as pl
from jax.experimental.pallas import tpu as pltpu
from jax.experimental.pallas import tpu_sc as plsc

N = 10000
E = 320000
D = 128
EMB = 128
L = 5
DE = 16
C = 10
G = 128

NW = 32
ROW = 128
RPW = 80
EPAD = NW * RPW * ROW
NITER = RPW // 4
NPAD = 10008
CHUNK = 624
LCHUNK = NPAD - 15 * CHUNK

NB = 1000
EB = 4096


def _sc_agg_body(h_hbm, e_hbm, idx_hbm, out_hbm,
                 iA, iB, ebuf, hb0, hb1, agg_sh,
                 sem_e, sem_h0, sem_h1, sem_s0, sem_s1, sem_iA, sem_iB):
    s = lax.axis_index("s")
    c = lax.axis_index("c")
    w = c * 16 + s
    pbase = w * (2 * RPW)
    erow0 = w * (RPW * ROW)

    def _zb(i, carry):
        for j in range(8):
            ebuf[i, pl.ds(j * 16, 16)] = jnp.zeros((16,), jnp.float32)
        return carry
    lax.fori_loop(0, ROW, _zb, 0)
    base = s * CHUNK

    def _zero_rows(nrows):
        for off in range(0, nrows - ROW + 1, ROW):
            pltpu.sync_copy(ebuf, agg_sh.at[pl.ds(base + off, ROW)])
        rem = nrows % ROW
        if rem:
            pltpu.sync_copy(ebuf.at[pl.ds(0, rem)],
                            agg_sh.at[pl.ds(base + nrows - rem, rem)])

    @pl.when(s < 15)
    def _():
        _zero_rows(CHUNK)

    @pl.when(s == 15)
    def _():
        _zero_rows(LCHUNK)
    plsc.subcore_barrier()

    def _relu_add(dst_b, src_b):
        def _cb(i2, c2):
            i = i2 * 4
            for k in range(4):
                for j in range(8):
                    sl = pl.ds(j * 16, 16)
                    dst_b[i + k, sl] = jnp.maximum(
                        dst_b[i + k, sl] + src_b[i + k, sl], 0.0)
            return c2
        lax.fori_loop(0, ROW // 4, _cb, 0)

    pltpu.sync_copy(idx_hbm.at[pl.ds(pbase, 4)], iA)
    pltpu.async_copy(e_hbm.at[pl.ds(erow0, ROW)], ebuf, sem_e)
    pltpu.async_copy(h_hbm.at[iA.at[0]], hb0, sem_h0)

    def _quad(m, carry):
        a = 4 * m
        erow_a = erow0 + a * ROW
        pbA = pbase + 8 * m

        @pl.when(m > 0)
        def _():
            pltpu.make_async_copy(hb1, agg_sh.at[iB.at[3]], sem_s1).wait()
        pltpu.async_copy(idx_hbm.at[pl.ds(pbA + 4, 4)], iB, sem_iB)
        pltpu.async_copy(h_hbm.at[iA.at[1]], hb1, sem_h1)

        pltpu.make_async_copy(e_hbm.at[pl.ds(erow_a, ROW)], ebuf, sem_e).wait()
        pltpu.make_async_copy(h_hbm.at[iA.at[0]], hb0, sem_h0).wait()
        _relu_add(hb0, ebuf)
        pltpu.async_copy(e_hbm.at[pl.ds(erow_a + ROW, ROW)], ebuf, sem_e)
        pltpu.async_copy(hb0, agg_sh.at[iA.at[2]], sem_s0, add=True)

        pltpu.make_async_copy(e_hbm.at[pl.ds(erow_a + ROW, ROW)], ebuf,
                              sem_e).wait()
        pltpu.make_async_copy(h_hbm.at[iA.at[1]], hb1, sem_h1).wait()
        _relu_add(hb1, ebuf)
        pltpu.make_async_copy(hb0, agg_sh.at[iA.at[2]], sem_s0).wait()

        pltpu.make_async_copy(idx_hbm.at[pl.ds(pbA + 4, 4)], iB, sem_iB).wait()
        pltpu.async_copy(h_hbm.at[iB.at[0]], hb0, sem_h0)
        pltpu.async_copy(e_hbm.at[pl.ds(erow_a + 2 * ROW, ROW)], ebuf, sem_e)
        pltpu.async_copy(hb1, agg_sh.at[iA.at[3]], sem_s1, add=True)

        pltpu.make_async_copy(hb1, agg_sh.at[iA.at[3]], sem_s1).wait()

        @pl.when(m < NITER - 1)
        def _():
            pltpu.async_copy(idx_hbm.at[pl.ds(pbA + 8, 4)], iA, sem_iA)
        pltpu.async_copy(h_hbm.at[iB.at[1]], hb1, sem_h1)

        pltpu.make_async_copy(e_hbm.at[pl.ds(erow_a + 2 * ROW, ROW)], ebuf,
                              sem_e).wait()
        pltpu.make_async_copy(h_hbm.at[iB.at[0]], hb0, sem_h0).wait()
        _relu_add(hb0, ebuf)
        pltpu.async_copy(e_hbm.at[pl.ds(erow_a + 3 * ROW, ROW)], ebuf, sem_e)
        pltpu.async_copy(hb0, agg_sh.at[iB.at[2]], sem_s0, add=True)

        pltpu.make_async_copy(e_hbm.at[pl.ds(erow_a + 3 * ROW, ROW)], ebuf,
                              sem_e).wait()
        pltpu.make_async_copy(h_hbm.at[iB.at[1]], hb1, sem_h1).wait()
        _relu_add(hb1, ebuf)
        pltpu.make_async_copy(hb0, agg_sh.at[iB.at[2]], sem_s0).wait()

        @pl.when(m < NITER - 1)
        def _():
            pltpu.make_async_copy(idx_hbm.at[pl.ds(pbA + 8, 4)], iA,
                                  sem_iA).wait()
            pltpu.async_copy(h_hbm.at[iA.at[0]], hb0, sem_h0)
            pltpu.async_copy(e_hbm.at[pl.ds(erow_a + 4 * ROW, ROW)], ebuf,
                             sem_e)
        pltpu.async_copy(hb1, agg_sh.at[iB.at[3]], sem_s1, add=True)
        return carry
    lax.fori_loop(0, NITER, _quad, 0)
    pltpu.make_async_copy(hb1, agg_sh.at[iB.at[3]], sem_s1).wait()

    plsc.subcore_barrier()

    @pl.when(s < 15)
    def _():
        pltpu.sync_copy(agg_sh.at[pl.ds(base, CHUNK)],
                        out_hbm.at[c, pl.ds(base, CHUNK)])

    @pl.when(s == 15)
    def _():
        pltpu.sync_copy(agg_sh.at[pl.ds(base, LCHUNK)],
                        out_hbm.at[c, pl.ds(base, LCHUNK)])


def _make_sc_agg():
    return pl.kernel(
        _sc_agg_body,
        out_type=jax.ShapeDtypeStruct((2, NPAD, EMB), jnp.float32),
        mesh=plsc.VectorSubcoreMesh(core_axis_name="c", subcore_axis_name="s",
                                    num_cores=2),
        scratch_types=[
        pltpu.VMEM((4, ROW), jnp.int32),
        pltpu.VMEM((4, ROW), jnp.int32),
        pltpu.VMEM((ROW, EMB), jnp.float32),
        pltpu.VMEM((ROW, EMB), jnp.float32),
        pltpu.VMEM((ROW, EMB), jnp.float32),
        pltpu.VMEM_SHARED((NPAD, EMB), jnp.float32),
        pltpu.SemaphoreType.DMA,
        pltpu.SemaphoreType.DMA,
        pltpu.SemaphoreType.DMA,
        pltpu.SemaphoreType.DMA,
        pltpu.SemaphoreType.DMA,
        pltpu.SemaphoreType.DMA,
        pltpu.SemaphoreType.DMA,
    ],
    )


_sc_agg = _make_sc_agg()


def _enc_body(x_ref, w_ref, b_ref, g_ref, be_ref, o_ref):
    acc = jnp.dot(x_ref[...], w_ref[...], preferred_element_type=jnp.float32)
    o_ref[...] = g_ref[...] * (acc + b_ref[...]) + be_ref[...]


def _encode(x, W_enc, b_enc, g0, be0):
    return pl.pallas_call(
        _enc_body,
        grid=(N // NB,),
        in_specs=[
            pl.BlockSpec((NB, D), lambda i: (i, 0)),
            pl.BlockSpec((D, EMB), lambda i: (0, 0)),
            pl.BlockSpec((1, EMB), lambda i: (0, 0)),
            pl.BlockSpec((1, EMB), lambda i: (0, 0)),
            pl.BlockSpec((1, EMB), lambda i: (0, 0)),
        ],
        out_specs=pl.BlockSpec((NB, EMB), lambda i: (i, 0)),
        out_shape=jax.ShapeDtypeStruct((N, EMB), jnp.float32),
    )(x, W_enc, b_enc.reshape(1, EMB), g0.reshape(1, EMB), be0.reshape(1, EMB))


def _eemb_body(ea_ref, w_ref, b_ref, o_ref):
    acc = jnp.dot(ea_ref[...], w_ref[...], preferred_element_type=jnp.float32)
    o_ref[...] = acc + b_ref[...]


def _edge_embed(ea_pad, W_l, b_l):
    return pl.pallas_call(
        _eemb_body,
        grid=(EPAD // EB,),
        in_specs=[
            pl.BlockSpec((EB, DE), lambda i: (i, 0)),
            pl.BlockSpec((DE, EMB), lambda i: (0, 0)),
            pl.BlockSpec((1, EMB), lambda i: (0, 0)),
        ],
        out_specs=pl.BlockSpec((EB, EMB), lambda i: (i, 0)),
        out_shape=jax.ShapeDtypeStruct((EPAD, EMB), jnp.float32),
    )(ea_pad, W_l, b_l.reshape(1, EMB))


def _node_body(do_relu, h_ref, p0_ref, p1_ref, eps_ref, w1_ref, b1_ref,
               w2_ref, b2_ref, gl_ref, bl_ref, o_ref):
    t = (1.0 + eps_ref[0, 0]) * h_ref[...] + p0_ref[...] + p1_ref[...]
    u = jnp.maximum(
        jnp.dot(t, w1_ref[...], preferred_element_type=jnp.float32)
        + b1_ref[...], 0.0)
    v = jnp.dot(u, w2_ref[...], preferred_element_type=jnp.float32) + b2_ref[...]
    t = gl_ref[...] * v + bl_ref[...]
    if do_relu:
        t = jnp.maximum(t, 0.0)
    o_ref[...] = t


def _node_mlp(h, p0, p1, eps_l, W1_l, b1_l, W2_l, b2_l, gl_l, bl_l, do_relu):
    return pl.pallas_call(
        functools.partial(_node_body, do_relu),
        grid=(N // NB,),
        in_specs=[
            pl.BlockSpec((NB, EMB), lambda i: (i, 0)),
            pl.BlockSpec((NB, EMB), lambda i: (i, 0)),
            pl.BlockSpec((NB, EMB), lambda i: (i, 0)),
            pl.BlockSpec((1, 1), lambda i: (0, 0)),
            pl.BlockSpec((EMB, 2 * EMB), lambda i: (0, 0)),
            pl.BlockSpec((1, 2 * EMB), lambda i: (0, 0)),
            pl.BlockSpec((2 * EMB, EMB), lambda i: (0, 0)),
            pl.BlockSpec((1, EMB), lambda i: (0, 0)),
            pl.BlockSpec((1, EMB), lambda i: (0, 0)),
            pl.BlockSpec((1, EMB), lambda i: (0, 0)),
        ],
        out_specs=pl.BlockSpec((NB, EMB), lambda i: (i, 0)),
        out_shape=jax.ShapeDtypeStruct((N, EMB), jnp.float32),
    )(h, p0, p1, eps_l.reshape(1, 1), W1_l, b1_l.reshape(1, 2 * EMB),
      W2_l, b2_l.reshape(1, EMB), gl_l.reshape(1, EMB), bl_l.reshape(1, EMB))


def _readout_body(h_ref, b_ref, wp1_ref, bp1_ref, wp2_ref, bp2_ref, o_ref,
                  hg_ref):
    i = pl.program_id(0)

    @pl.when(i == 0)
    def _():
        hg_ref[...] = jnp.zeros_like(hg_ref)

    bblk = b_ref[0, 0, :]
    onehot = (lax.broadcasted_iota(jnp.int32, (G, NB), 0)
              == bblk[None, :]).astype(jnp.float32)
    hg_ref[...] += jnp.dot(onehot, h_ref[...],
                           preferred_element_type=jnp.float32)

    @pl.when(i == (N // NB) - 1)
    def _():
        hg = hg_ref[...]
        z = jax.nn.sigmoid(
            jnp.dot(hg, wp1_ref[...], preferred_element_type=jnp.float32)
            + bp1_ref[...])
        o_ref[...] = jnp.dot(z, wp2_ref[...],
                             preferred_element_type=jnp.float32) + bp2_ref[...]


def _readout(h, batch3d, Wp1, bp1, Wp2, bp2):
    return pl.pallas_call(
        _readout_body,
        grid=(N // NB,),
        in_specs=[
            pl.BlockSpec((NB, EMB), lambda i: (i, 0)),
            pl.BlockSpec((1, 1, NB), lambda i: (i, 0, 0)),
            pl.BlockSpec((EMB, EMB), lambda i: (0, 0)),
            pl.BlockSpec((1, EMB), lambda i: (0, 0)),
            pl.BlockSpec((EMB, C), lambda i: (0, 0)),
            pl.BlockSpec((1, C), lambda i: (0, 0)),
        ],
        out_specs=pl.BlockSpec((G, C), lambda i: (0, 0)),
        out_shape=jax.ShapeDtypeStruct((G, C), jnp.float32),
        scratch_shapes=[pltpu.VMEM((G, EMB), jnp.float32)],
    )(h, batch3d, Wp1, bp1.reshape(1, EMB), Wp2, bp2.reshape(1, C))


def kernel(x, edge_index, edge_attr, batch, W_enc, b_enc, g0, be0,
           W_edge, b_edge, eps, W1, b1, W2, b2, gl, bl, Wp1, bp1, Wp2, bp2):
    pad = EPAD - E
    src2d = jnp.concatenate(
        [edge_index[0], jnp.zeros((pad,), jnp.int32)]).reshape(EPAD // ROW, ROW)
    dst2d = jnp.concatenate(
        [edge_index[1], jnp.full((pad,), N, jnp.int32)]).reshape(EPAD // ROW, ROW)
    npair_all = EPAD // ROW // 2
    idx_packed = jnp.concatenate(
        [src2d.reshape(npair_all, 2, ROW), dst2d.reshape(npair_all, 2, ROW)],
        axis=1).reshape(4 * npair_all, ROW)
    ea_pad = jnp.concatenate(
        [edge_attr, jnp.zeros((pad, DE), jnp.float32)])
    batch3d = batch.reshape(N // NB, 1, NB)

    h = _encode(x, W_enc, b_enc, g0, be0)
    e_embs = [_edge_embed(ea_pad, W_edge[l], b_edge[l]) for l in range(L)]
    for l in range(L):
        p = _sc_agg(h, e_embs[l], idx_packed)
        h = _node_mlp(h, p[0], p[1], eps[l], W1[l], b1[l], W2[l], b2[l],
                      gl[l], bl[l], do_relu=(l < L - 1))
    return _readout(h, batch3d, Wp1, bp1, Wp2, bp2)


# speedup vs baseline: 1.3122x; 1.0272x over previous
import functools

import jax
import jax.numpy as jnp
from jax import lax
from jax.experimental import pallas as pl
from jax.experimental.pallas import tpu as pltpu
from jax.experimental.pallas import tpu_sc as plsc

N = 10000
E = 320000
D = 128
EMB = 128
L = 5
DE = 16
C = 10
G = 128

NW = 32
ROW = 128
RA = 96
RB = 64
EPAD = 16 * (RA + RB) * ROW
NPAD = 10008
CHUNK = 624
LCHUNK = NPAD - 15 * CHUNK

NB = 1000
EB = 4096


def _sc_agg_body(h_hbm, e_hbm, idx_hbm, out_hbm,
                 iA, iB, ebuf, hb0, hb1, agg_sh,
                 sem_e, sem_h0, sem_h1, sem_s0, sem_s1, sem_iA, sem_iB):
    s = lax.axis_index("s")
    c = lax.axis_index("c")
    row0 = jnp.where(c == 0, s * RA, 16 * RA + s * RB)
    pbase = 2 * row0
    erow0 = row0 * ROW
    nit = jnp.where(c == 0, RA // 4, RB // 4)

    def _zb(i, carry):
        for j in range(8):
            ebuf[i, pl.ds(j * 16, 16)] = jnp.zeros((16,), jnp.float32)
        return carry
    lax.fori_loop(0, ROW, _zb, 0)
    base = s * CHUNK

    def _zero_rows(nrows):
        for off in range(0, nrows - ROW + 1, ROW):
            pltpu.sync_copy(ebuf, agg_sh.at[pl.ds(base + off, ROW)])
        rem = nrows % ROW
        if rem:
            pltpu.sync_copy(ebuf.at[pl.ds(0, rem)],
                            agg_sh.at[pl.ds(base + nrows - rem, rem)])

    @pl.when(s < 15)
    def _():
        _zero_rows(CHUNK)

    @pl.when(s == 15)
    def _():
        _zero_rows(LCHUNK)
    plsc.subcore_barrier()

    def _relu_add(dst_b, src_b):
        def _cb(i2, c2):
            i = i2 * 4
            for k in range(4):
                for j in range(8):
                    sl = pl.ds(j * 16, 16)
                    dst_b[i + k, sl] = jnp.maximum(
                        dst_b[i + k, sl] + src_b[i + k, sl], 0.0)
            return c2
        lax.fori_loop(0, ROW // 4, _cb, 0)

    pltpu.sync_copy(idx_hbm.at[pl.ds(pbase, 4)], iA)
    pltpu.async_copy(e_hbm.at[pl.ds(erow0, ROW)], ebuf, sem_e)
    pltpu.async_copy(h_hbm.at[iA.at[0]], hb0, sem_h0)

    def _quad(m, carry):
      @pl.when(m < nit)
      def _():
        a = 4 * m
        erow_a = erow0 + a * ROW
        pbA = pbase + 8 * m

        @pl.when(m > 0)
        def _():
            pltpu.make_async_copy(hb1, agg_sh.at[iB.at[3]], sem_s1).wait()
        pltpu.async_copy(idx_hbm.at[pl.ds(pbA + 4, 4)], iB, sem_iB)
        pltpu.async_copy(h_hbm.at[iA.at[1]], hb1, sem_h1)

        pltpu.make_async_copy(e_hbm.at[pl.ds(erow_a, ROW)], ebuf, sem_e).wait()
        pltpu.make_async_copy(h_hbm.at[iA.at[0]], hb0, sem_h0).wait()
        _relu_add(hb0, ebuf)
        pltpu.async_copy(e_hbm.at[pl.ds(erow_a + ROW, ROW)], ebuf, sem_e)
        pltpu.async_copy(hb0, agg_sh.at[iA.at[2]], sem_s0, add=True)

        pltpu.make_async_copy(e_hbm.at[pl.ds(erow_a + ROW, ROW)], ebuf,
                              sem_e).wait()
        pltpu.make_async_copy(h_hbm.at[iA.at[1]], hb1, sem_h1).wait()
        _relu_add(hb1, ebuf)
        pltpu.make_async_copy(hb0, agg_sh.at[iA.at[2]], sem_s0).wait()

        pltpu.make_async_copy(idx_hbm.at[pl.ds(pbA + 4, 4)], iB, sem_iB).wait()
        pltpu.async_copy(h_hbm.at[iB.at[0]], hb0, sem_h0)
        pltpu.async_copy(e_hbm.at[pl.ds(erow_a + 2 * ROW, ROW)], ebuf, sem_e)
        pltpu.async_copy(hb1, agg_sh.at[iA.at[3]], sem_s1, add=True)

        pltpu.make_async_copy(hb1, agg_sh.at[iA.at[3]], sem_s1).wait()

        @pl.when(m < nit - 1)
        def _():
            pltpu.async_copy(idx_hbm.at[pl.ds(pbA + 8, 4)], iA, sem_iA)
        pltpu.async_copy(h_hbm.at[iB.at[1]], hb1, sem_h1)

        pltpu.make_async_copy(e_hbm.at[pl.ds(erow_a + 2 * ROW, ROW)], ebuf,
                              sem_e).wait()
        pltpu.make_async_copy(h_hbm.at[iB.at[0]], hb0, sem_h0).wait()
        _relu_add(hb0, ebuf)
        pltpu.async_copy(e_hbm.at[pl.ds(erow_a + 3 * ROW, ROW)], ebuf, sem_e)
        pltpu.async_copy(hb0, agg_sh.at[iB.at[2]], sem_s0, add=True)

        pltpu.make_async_copy(e_hbm.at[pl.ds(erow_a + 3 * ROW, ROW)], ebuf,
                              sem_e).wait()
        pltpu.make_async_copy(h_hbm.at[iB.at[1]], hb1, sem_h1).wait()
        _relu_add(hb1, ebuf)
        pltpu.make_async_copy(hb0, agg_sh.at[iB.at[2]], sem_s0).wait()

        @pl.when(m < nit - 1)
        def _():
            pltpu.make_async_copy(idx_hbm.at[pl.ds(pbA + 8, 4)], iA,
                                  sem_iA).wait()
            pltpu.async_copy(h_hbm.at[iA.at[0]], hb0, sem_h0)
            pltpu.async_copy(e_hbm.at[pl.ds(erow_a + 4 * ROW, ROW)], ebuf,
                             sem_e)
        pltpu.async_copy(hb1, agg_sh.at[iB.at[3]], sem_s1, add=True)
      return carry
    lax.fori_loop(0, RA // 4, _quad, 0)
    pltpu.make_async_copy(hb1, agg_sh.at[iB.at[3]], sem_s1).wait()

    plsc.subcore_barrier()

    @pl.when(s < 15)
    def _():
        pltpu.sync_copy(agg_sh.at[pl.ds(base, CHUNK)],
                        out_hbm.at[c, pl.ds(base, CHUNK)])

    @pl.when(s == 15)
    def _():
        pltpu.sync_copy(agg_sh.at[pl.ds(base, LCHUNK)],
                        out_hbm.at[c, pl.ds(base, LCHUNK)])


def _make_sc_agg():
    return pl.kernel(
        _sc_agg_body,
        out_type=jax.ShapeDtypeStruct((2, NPAD, EMB), jnp.float32),
        mesh=plsc.VectorSubcoreMesh(core_axis_name="c", subcore_axis_name="s",
                                    num_cores=2),
        scratch_types=[
        pltpu.VMEM((4, ROW), jnp.int32),
        pltpu.VMEM((4, ROW), jnp.int32),
        pltpu.VMEM((ROW, EMB), jnp.float32),
        pltpu.VMEM((ROW, EMB), jnp.float32),
        pltpu.VMEM((ROW, EMB), jnp.float32),
        pltpu.VMEM_SHARED((NPAD, EMB), jnp.float32),
        pltpu.SemaphoreType.DMA,
        pltpu.SemaphoreType.DMA,
        pltpu.SemaphoreType.DMA,
        pltpu.SemaphoreType.DMA,
        pltpu.SemaphoreType.DMA,
        pltpu.SemaphoreType.DMA,
        pltpu.SemaphoreType.DMA,
    ],
    )


_sc_agg = _make_sc_agg()


def _enc_body(x_ref, w_ref, b_ref, g_ref, be_ref, o_ref):
    acc = jnp.dot(x_ref[...], w_ref[...], preferred_element_type=jnp.float32)
    o_ref[...] = g_ref[...] * (acc + b_ref[...]) + be_ref[...]


def _encode(x, W_enc, b_enc, g0, be0):
    return pl.pallas_call(
        _enc_body,
        grid=(N // NB,),
        in_specs=[
            pl.BlockSpec((NB, D), lambda i: (i, 0)),
            pl.BlockSpec((D, EMB), lambda i: (0, 0)),
            pl.BlockSpec((1, EMB), lambda i: (0, 0)),
            pl.BlockSpec((1, EMB), lambda i: (0, 0)),
            pl.BlockSpec((1, EMB), lambda i: (0, 0)),
        ],
        out_specs=pl.BlockSpec((NB, EMB), lambda i: (i, 0)),
        out_shape=jax.ShapeDtypeStruct((N, EMB), jnp.float32),
    )(x, W_enc, b_enc.reshape(1, EMB), g0.reshape(1, EMB), be0.reshape(1, EMB))


def _eemb_body(ea_ref, w_ref, b_ref, o_ref):
    acc = jnp.dot(ea_ref[...], w_ref[...], preferred_element_type=jnp.float32)
    o_ref[...] = acc + b_ref[...]


def _edge_embed(ea_pad, W_l, b_l):
    return pl.pallas_call(
        _eemb_body,
        grid=(EPAD // EB,),
        in_specs=[
            pl.BlockSpec((EB, DE), lambda i: (i, 0)),
            pl.BlockSpec((DE, EMB), lambda i: (0, 0)),
            pl.BlockSpec((1, EMB), lambda i: (0, 0)),
        ],
        out_specs=pl.BlockSpec((EB, EMB), lambda i: (i, 0)),
        out_shape=jax.ShapeDtypeStruct((EPAD, EMB), jnp.float32),
    )(ea_pad, W_l, b_l.reshape(1, EMB))


def _node_body(do_relu, h_ref, p0_ref, p1_ref, eps_ref, w1_ref, b1_ref,
               w2_ref, b2_ref, gl_ref, bl_ref, o_ref):
    t = (1.0 + eps_ref[0, 0]) * h_ref[...] + p0_ref[...] + p1_ref[...]
    u = jnp.maximum(
        jnp.dot(t, w1_ref[...], preferred_element_type=jnp.float32)
        + b1_ref[...], 0.0)
    v = jnp.dot(u, w2_ref[...], preferred_element_type=jnp.float32) + b2_ref[...]
    t = gl_ref[...] * v + bl_ref[...]
    if do_relu:
        t = jnp.maximum(t, 0.0)
    o_ref[...] = t


def _node_mlp(h, p0, p1, eps_l, W1_l, b1_l, W2_l, b2_l, gl_l, bl_l, do_relu):
    return pl.pallas_call(
        functools.partial(_node_body, do_relu),
        grid=(N // NB,),
        in_specs=[
            pl.BlockSpec((NB, EMB), lambda i: (i, 0)),
            pl.BlockSpec((NB, EMB), lambda i: (i, 0)),
            pl.BlockSpec((NB, EMB), lambda i: (i, 0)),
            pl.BlockSpec((1, 1), lambda i: (0, 0)),
            pl.BlockSpec((EMB, 2 * EMB), lambda i: (0, 0)),
            pl.BlockSpec((1, 2 * EMB), lambda i: (0, 0)),
            pl.BlockSpec((2 * EMB, EMB), lambda i: (0, 0)),
            pl.BlockSpec((1, EMB), lambda i: (0, 0)),
            pl.BlockSpec((1, EMB), lambda i: (0, 0)),
            pl.BlockSpec((1, EMB), lambda i: (0, 0)),
        ],
        out_specs=pl.BlockSpec((NB, EMB), lambda i: (i, 0)),
        out_shape=jax.ShapeDtypeStruct((N, EMB), jnp.float32),
    )(h, p0, p1, eps_l.reshape(1, 1), W1_l, b1_l.reshape(1, 2 * EMB),
      W2_l, b2_l.reshape(1, EMB), gl_l.reshape(1, EMB), bl_l.reshape(1, EMB))


def _readout_body(h_ref, b_ref, wp1_ref, bp1_ref, wp2_ref, bp2_ref, o_ref,
                  hg_ref):
    i = pl.program_id(0)

    @pl.when(i == 0)
    def _():
        hg_ref[...] = jnp.zeros_like(hg_ref)

    bblk = b_ref[0, 0, :]
    onehot = (lax.broadcasted_iota(jnp.int32, (G, NB), 0)
              == bblk[None, :]).astype(jnp.float32)
    hg_ref[...] += jnp.dot(onehot, h_ref[...],
                           preferred_element_type=jnp.float32)

    @pl.when(i == (N // NB) - 1)
    def _():
        hg = hg_ref[...]
        z = jax.nn.sigmoid(
            jnp.dot(hg, wp1_ref[...], preferred_element_type=jnp.float32)
            + bp1_ref[...])
        o_ref[...] = jnp.dot(z, wp2_ref[...],
                             preferred_element_type=jnp.float32) + bp2_ref[...]


def _readout(h, batch3d, Wp1, bp1, Wp2, bp2):
    return pl.pallas_call(
        _readout_body,
        grid=(N // NB,),
        in_specs=[
            pl.BlockSpec((NB, EMB), lambda i: (i, 0)),
            pl.BlockSpec((1, 1, NB), lambda i: (i, 0, 0)),
            pl.BlockSpec((EMB, EMB), lambda i: (0, 0)),
            pl.BlockSpec((1, EMB), lambda i: (0, 0)),
            pl.BlockSpec((EMB, C), lambda i: (0, 0)),
            pl.BlockSpec((1, C), lambda i: (0, 0)),
        ],
        out_specs=pl.BlockSpec((G, C), lambda i: (0, 0)),
        out_shape=jax.ShapeDtypeStruct((G, C), jnp.float32),
        scratch_shapes=[pltpu.VMEM((G, EMB), jnp.float32)],
    )(h, batch3d, Wp1, bp1.reshape(1, EMB), Wp2, bp2.reshape(1, C))


def kernel(x, edge_index, edge_attr, batch, W_enc, b_enc, g0, be0,
           W_edge, b_edge, eps, W1, b1, W2, b2, gl, bl, Wp1, bp1, Wp2, bp2):
    pad = EPAD - E
    src2d = jnp.concatenate(
        [edge_index[0], jnp.zeros((pad,), jnp.int32)]).reshape(EPAD // ROW, ROW)
    dst2d = jnp.concatenate(
        [edge_index[1], jnp.full((pad,), N, jnp.int32)]).reshape(EPAD // ROW, ROW)
    npair_all = EPAD // ROW // 2
    idx_packed = jnp.concatenate(
        [src2d.reshape(npair_all, 2, ROW), dst2d.reshape(npair_all, 2, ROW)],
        axis=1).reshape(4 * npair_all, ROW)
    ea_pad = jnp.concatenate(
        [edge_attr, jnp.zeros((pad, DE), jnp.float32)])
    batch3d = batch.reshape(N // NB, 1, NB)

    h = _encode(x, W_enc, b_enc, g0, be0)
    e_embs = [_edge_embed(ea_pad, W_edge[l], b_edge[l]) for l in range(L)]
    for l in range(L):
        p = _sc_agg(h, e_embs[l], idx_packed)
        h = _node_mlp(h, p[0], p[1], eps[l], W1[l], b1[l], W2[l], b2[l],
                      gl[l], bl[l], do_relu=(l < L - 1))
    return _readout(h, batch3d, Wp1, bp1, Wp2, bp2)
